# baseline (speedup 1.0000x reference)
import functools

import jax
import jax.numpy as jnp
from jax import lax
from jax.experimental import pallas as pl
from jax.experimental.pallas import tpu as pltpu
from jax.experimental.pallas import tpu_sc as plsc

BATCH = 16384
EMBED_DIM = 64
SHARED_DIM = 16
NUM_WORKERS = 32
B_PER_W = BATCH // NUM_WORKERS
CHUNK = 256
N_CHUNKS = B_PER_W // CHUNK
TOP_GROUPS = 100624
BOT_GROUPS = 125000 - TOP_GROUPS


def _sc_kernel():
    mesh = plsc.VectorSubcoreMesh(core_axis_name="c", subcore_axis_name="s")

    @functools.partial(
        pl.kernel,
        out_type=jax.ShapeDtypeStruct((EMBED_DIM, BATCH), jnp.float32),
        mesh=mesh,
        scratch_types=[
            pltpu.VMEM((B_PER_W + 16,), jnp.int32),
            pltpu.VMEM((B_PER_W, EMBED_DIM), jnp.float32),
            pltpu.VMEM((EMBED_DIM, B_PER_W), jnp.float32),
            pltpu.VMEM((SHARED_DIM,), jnp.float32),
            pltpu.SemaphoreType.DMA,
            pltpu.SemaphoreType.DMA,
        ],
        compiler_params=pltpu.CompilerParams(needs_layout_passes=False),
    )
    def k(x_hbm, w_hbm, wb_hbm, sh_hbm, out_hbm, idx_v, rbuf_v, trans_v,
          sh_v, gsem, osem):
        wid = lax.axis_index("s") * 2 + lax.axis_index("c")
        base = wid * B_PER_W
        pltpu.sync_copy(x_hbm.at[pl.ds(base, B_PER_W)],
                        idx_v.at[pl.ds(0, B_PER_W)])
        pltpu.sync_copy(sh_hbm.at[0], sh_v)
        sh = sh_v[...]

        def issue_chunk(c, sem):
            def issue(r, carry):
                g = c * CHUNK + r
                v = idx_v[pl.ds(g, 16)]
                i = v[0]
                t = lax.shift_right_logical(i, 3)
                a = i & 7

                @pl.when(t < TOP_GROUPS)
                def _():
                    pltpu.async_copy(w_hbm.at[t, a], rbuf_v.at[g], sem)

                @pl.when(t >= TOP_GROUPS)
                def _():
                    pltpu.async_copy(wb_hbm.at[i - TOP_GROUPS * 8],
                                     rbuf_v.at[g], sem)

                return carry

            lax.fori_loop(0, CHUNK, issue, 0, unroll=4)

        def drain_chunk(c, sem):
            pltpu.make_async_copy(w_hbm.at[pl.ds(0, CHUNK), 0],
                                  rbuf_v.at[pl.ds(c * CHUNK, CHUNK)],
                                  sem).wait()

        def transpose_chunk(c):
            def transpose(rb, carry):
                iota = lax.iota(jnp.int32, 16)
                r0 = (c * (CHUNK // 16)
                      + lax.convert_element_type(rb, jnp.int32)) * 16
                rvec = iota + r0
                for kk in range(3):
                    d0 = SHARED_DIM + 16 * kk
                    for kq in range(16):
                        dvec = ((iota + kq) & 15) + d0
                        x = plsc.load_gather(rbuf_v, [rvec, dvec])
                        plsc.store_scatter(trans_v, [dvec, rvec], x)
                return carry

            lax.fori_loop(0, CHUNK // 16, transpose, 0)

        issue_chunk(0, gsem)
        issue_chunk(1, osem)
        for d in range(SHARED_DIM):
            val = lax.broadcast_in_dim(sh[d], (16,), ())
            for kk in range(B_PER_W // 16):
                trans_v[d, pl.ds(kk * 16, 16)] = val
        for c in range(N_CHUNKS):
            sem = gsem if c % 2 == 0 else osem
            drain_chunk(c, sem)
            if c + 2 < N_CHUNKS:
                issue_chunk(c + 2, sem)
            transpose_chunk(c)

        pltpu.async_copy(trans_v, out_hbm.at[:, pl.ds(base, B_PER_W)],
                         osem).wait()

    return k


_k = _sc_kernel()


def kernel(X, W, shared_embed):
    W_top = W[:TOP_GROUPS * 8].reshape(TOP_GROUPS, 8, EMBED_DIM)
    W_bot = W[TOP_GROUPS * 8:]
    out_t = _k(X.astype(jnp.int32), W_top, W_bot, shared_embed)
    return out_t.T


# speedup vs baseline: 1.5952x; 1.5952x over previous
import functools

import jax
import jax.numpy as jnp
from jax import lax
from jax.experimental import pallas as pl
from jax.experimental.pallas import tpu as pltpu
from jax.experimental.pallas import tpu_sc as plsc

BATCH = 16384
EMBED_DIM = 64
SHARED_DIM = 16
NUM_WORKERS = 32
B_PER_W = BATCH // NUM_WORKERS


def _sc_kernel():
    mesh = plsc.VectorSubcoreMesh(core_axis_name="c", subcore_axis_name="s")

    @functools.partial(
        pl.kernel,
        out_type=jax.ShapeDtypeStruct((EMBED_DIM, BATCH), jnp.float32),
        mesh=mesh,
        scratch_types=[
            pltpu.VMEM((B_PER_W + 16,), jnp.int32),
            pltpu.VMEM((B_PER_W, EMBED_DIM), jnp.float32),
            pltpu.VMEM((EMBED_DIM, B_PER_W), jnp.float32),
            pltpu.VMEM((SHARED_DIM,), jnp.float32),
            pltpu.SemaphoreType.DMA,
            pltpu.SemaphoreType.DMA,
        ],
        compiler_params=pltpu.CompilerParams(needs_layout_passes=False),
    )
    def k(x_hbm, w_hbm, sh_hbm, out_hbm, idx_v, rbuf_v, trans_v, sh_v,
          gsem, osem):
        wid = lax.axis_index("s") * 2 + lax.axis_index("c")
        base = wid * B_PER_W
        pltpu.sync_copy(x_hbm.at[pl.ds(base, B_PER_W)],
                        idx_v.at[pl.ds(0, B_PER_W)])
        pltpu.sync_copy(sh_hbm.at[0], sh_v)
        sh = sh_v[...]

        def issue(r, carry):
            v = idx_v[pl.ds(r, 16)]
            i = v[0]
            t = lax.shift_right_logical(i, 3)
            a = i & 7
            pltpu.async_copy(w_hbm.at[t, a], rbuf_v.at[r], gsem)
            return carry

        lax.fori_loop(0, B_PER_W, issue, 0, unroll=4)

        for d in range(SHARED_DIM):
            val = lax.broadcast_in_dim(sh[d], (16,), ())
            for kk in range(B_PER_W // 16):
                trans_v[d, pl.ds(kk * 16, 16)] = val

        pltpu.make_async_copy(w_hbm.at[pl.ds(0, B_PER_W), 0], rbuf_v,
                              gsem).wait()

        iota = lax.iota(jnp.int32, 16)
        perms = [(iota + kq) & 15 for kq in range(16)]

        def transpose(rb, carry):
            r0 = lax.convert_element_type(rb, jnp.int32) * 16
            rvec = iota + r0
            for kk in range(3):
                d0 = SHARED_DIM + 16 * kk
                for kq in range(16):
                    dvec = perms[kq] + d0
                    x = plsc.load_gather(rbuf_v, [rvec, dvec])
                    plsc.store_scatter(trans_v, [dvec, rvec], x)
            return carry

        lax.fori_loop(0, B_PER_W // 16, transpose, 0)

        pltpu.async_copy(trans_v, out_hbm.at[:, pl.ds(base, B_PER_W)],
                         osem).wait()

    return k


_k = _sc_kernel()


def kernel(X, W, shared_embed):
    W8 = W.reshape(125000, 8, EMBED_DIM)
    out_t = _k(X.astype(jnp.int32), W8, shared_embed)
    return out_t.T


# speedup vs baseline: 1.6293x; 1.0214x over previous
import functools

import jax
import jax.numpy as jnp
from jax import lax
from jax.experimental import pallas as pl
from jax.experimental.pallas import tpu as pltpu
from jax.experimental.pallas import tpu_sc as plsc

BATCH = 16384
EMBED_DIM = 64
SHARED_DIM = 16
NUM_WORKERS = 32
B_PER_W = BATCH // NUM_WORKERS


def _sc_kernel():
    mesh = plsc.VectorSubcoreMesh(core_axis_name="c", subcore_axis_name="s")

    @functools.partial(
        pl.kernel,
        out_type=jax.ShapeDtypeStruct((EMBED_DIM, BATCH), jnp.float32),
        mesh=mesh,
        scratch_types=[
            pltpu.VMEM((B_PER_W + 16,), jnp.int32),
            pltpu.VMEM((B_PER_W, EMBED_DIM), jnp.float32),
            pltpu.VMEM((EMBED_DIM, B_PER_W), jnp.float32),
            pltpu.VMEM((SHARED_DIM,), jnp.float32),
            pltpu.SemaphoreType.DMA,
            pltpu.SemaphoreType.DMA,
        ],
        compiler_params=pltpu.CompilerParams(needs_layout_passes=False),
    )
    def k(x_hbm, w_hbm, sh_hbm, out_hbm, idx_v, rbuf_v, trans_v, sh_v,
          gsem, osem):
        wid = lax.axis_index("s") * 2 + lax.axis_index("c")
        base = wid * B_PER_W
        pltpu.sync_copy(x_hbm.at[pl.ds(base, B_PER_W)],
                        idx_v.at[pl.ds(0, B_PER_W)])
        pltpu.sync_copy(sh_hbm.at[0], sh_v)
        sh = sh_v[...]

        def issue(q, carry):
            g = q * 4
            v = idx_v[pl.ds(g, 16)]
            for j in range(4):
                i = v[j]
                t = lax.shift_right_logical(i, 3)
                a = i & 7
                pltpu.async_copy(w_hbm.at[t, a], rbuf_v.at[g + j], gsem)
            return carry

        lax.fori_loop(0, B_PER_W // 4, issue, 0, unroll=2)

        for d in range(SHARED_DIM):
            val = lax.broadcast_in_dim(sh[d], (16,), ())
            for kk in range(B_PER_W // 16):
                trans_v[d, pl.ds(kk * 16, 16)] = val

        pltpu.make_async_copy(w_hbm.at[pl.ds(0, B_PER_W), 0], rbuf_v,
                              gsem).wait()

        iota = lax.iota(jnp.int32, 16)
        perms = [(iota + kq) & 15 for kq in range(16)]

        def transpose(rb, carry):
            r0 = lax.convert_element_type(rb, jnp.int32) * 16
            rvec = iota + r0
            for kk in range(3):
                d0 = SHARED_DIM + 16 * kk
                for kq in range(16):
                    dvec = perms[kq] + d0
                    x = plsc.load_gather(rbuf_v, [rvec, dvec])
                    plsc.store_scatter(trans_v, [dvec, rvec], x)
            return carry

        lax.fori_loop(0, B_PER_W // 16, transpose, 0)

        pltpu.async_copy(trans_v, out_hbm.at[:, pl.ds(base, B_PER_W)],
                         osem).wait()

    return k


_k = _sc_kernel()


def kernel(X, W, shared_embed):
    W8 = W.reshape(125000, 8, EMBED_DIM)
    out_t = _k(X.astype(jnp.int32), W8, shared_embed)
    return out_t.T
